# Initial kernel scaffold; baseline (speedup 1.0000x reference)
import jax
import jax.numpy as jnp
from jax.experimental import pallas as pl


def kernel(x, gate_w, w1, w2, w3):
    raise NotImplementedError("write your pallas kernel here")


# speedup vs baseline: 2.1896x; 2.1896x over previous
import jax
import jax.numpy as jnp
from jax.experimental import pallas as pl
from jax.experimental.pallas import tpu as pltpu

D = 2048
H = 512
E = 64
N = 32


def _gate_kernel(x_ref, gw_ref, se_ref, st_ref, sp_ref):
    x = x_ref[...]
    gw = gw_ref[...]
    s = jax.lax.dot_general(x, gw, (((1,), (1,)), ((), ())),
                            preferred_element_type=jnp.float32)
    m = jnp.max(s, axis=1, keepdims=True)
    p_top = 1.0 / jnp.sum(jnp.exp(s - m), axis=1, keepdims=True)
    col = jax.lax.broadcasted_iota(jnp.int32, (N, E), 1)
    e_id = jnp.min(jnp.where(s == m, col, E), axis=1)

    ei = e_id[:, None]
    ej = e_id[None, :]
    ii = jax.lax.broadcasted_iota(jnp.int32, (N, N), 0)
    jj = jax.lax.broadcasted_iota(jnp.int32, (N, N), 1)
    before = (ej < ei) | ((ej == ei) & (jj < ii))
    rank = jnp.sum(before.astype(jnp.int32), axis=1)

    kk = jax.lax.broadcasted_iota(jnp.int32, (N, N), 0)
    P = (rank[None, :] == kk)
    sorted_e = jnp.sum(jnp.where(P, e_id[None, :], 0), axis=1)
    tok = jax.lax.broadcasted_iota(jnp.int32, (N, N), 1)
    sorted_t = jnp.sum(jnp.where(P, tok, 0), axis=1)
    sorted_p = jnp.sum(jnp.where(P, p_top[:, 0][None, :], 0.0), axis=1)

    se_ref[0, :] = sorted_e
    st_ref[0, :] = sorted_t
    sp_ref[0, :] = sorted_p


def _ffn_kernel(se_ref, st_ref, x_ref, w1_ref, w3_ref, w2_ref, p_ref, o_ref):
    del se_ref, st_ref
    x = x_ref[0]
    w1 = w1_ref[0]
    w3 = w3_ref[0]
    w2 = w2_ref[0]
    h1 = jax.lax.dot_general(x, w1, (((1,), (1,)), ((), ())),
                             preferred_element_type=jnp.float32)
    h3 = jax.lax.dot_general(x, w3, (((1,), (1,)), ((), ())),
                             preferred_element_type=jnp.float32)
    h = jax.nn.silu(h1) * h3
    y = jax.lax.dot_general(h, w2, (((1,), (1,)), ((), ())),
                            preferred_element_type=jnp.float32)
    o_ref[0] = y * p_ref[0, 0, 0]


def kernel(x, gate_w, w1, w2, w3):
    orig_shape = x.shape
    xf = x.reshape(-1, orig_shape[-1])

    se, st, sp = pl.pallas_call(
        _gate_kernel,
        out_shape=(
            jax.ShapeDtypeStruct((1, N), jnp.int32),
            jax.ShapeDtypeStruct((1, N), jnp.int32),
            jax.ShapeDtypeStruct((1, N), jnp.float32),
        ),
    )(xf, gate_w)

    se1 = se.reshape(N)
    st1 = st.reshape(N)
    spv = sp.reshape(N, 1, 1)
    x3 = xf.reshape(N, 1, D)

    grid_spec = pltpu.PrefetchScalarGridSpec(
        num_scalar_prefetch=2,
        grid=(N,),
        in_specs=[
            pl.BlockSpec((1, 1, D), lambda i, se_r, st_r: (st_r[i], 0, 0)),
            pl.BlockSpec((1, H, D), lambda i, se_r, st_r: (se_r[i], 0, 0)),
            pl.BlockSpec((1, H, D), lambda i, se_r, st_r: (se_r[i], 0, 0)),
            pl.BlockSpec((1, D, H), lambda i, se_r, st_r: (se_r[i], 0, 0)),
            pl.BlockSpec((1, 1, 1), lambda i, se_r, st_r: (i, 0, 0)),
        ],
        out_specs=pl.BlockSpec((1, 1, D), lambda i, se_r, st_r: (st_r[i], 0, 0)),
    )

    y = pl.pallas_call(
        _ffn_kernel,
        grid_spec=grid_spec,
        out_shape=jax.ShapeDtypeStruct((N, 1, D), jnp.float32),
        compiler_params=pltpu.CompilerParams(
            vmem_limit_bytes=100 * 1024 * 1024,
        ),
    )(se1, st1, x3, w1, w3, w2, spv)

    return y.reshape(orig_shape)
